# baseline (speedup 1.0000x reference)
import functools

import jax
import jax.numpy as jnp
from jax import lax
from jax.experimental import pallas as pl
from jax.experimental.pallas import tpu as pltpu
from jax.experimental.pallas import tpu_sc as plsc

_NUM_CORES = 2
_NUM_SUBCORES = 16
_NUM_WORKERS = _NUM_CORES * _NUM_SUBCORES

_CB = 112
_N_CHUNKS = 448
_N_PAD = _CB * _N_CHUNKS
_CHUNKS_PER_WORKER = _N_CHUNKS // _NUM_WORKERS
_LANES = 16

_SPLIT = 5


def _matmul_tables(features, wmat, bvec):
  n, nin = features.shape
  kout = wmat.shape[1]
  bn = 5000
  assert n % bn == 0

  f_vol = kout // nin

  def body(x_ref, w_ref, b_ref, t_ref):
    acc = (
        jnp.dot(x_ref[...].astype(jnp.bfloat16), w_ref[...],
                preferred_element_type=jnp.float32)
        + b_ref[...])
    for f in range(f_vol):
      t_ref[f] = acc[:, f * nin:(f + 1) * nin]

  return pl.pallas_call(
      body,
      grid=(n // bn,),
      in_specs=[
          pl.BlockSpec((bn, nin), lambda i: (i, 0)),
          pl.BlockSpec((nin, kout), lambda i: (0, 0)),
          pl.BlockSpec((1, kout), lambda i: (0, 0)),
      ],
      out_specs=pl.BlockSpec((f_vol, bn, nin), lambda i: (0, i, 0)),
      out_shape=jax.ShapeDtypeStruct((f_vol, n, nin), jnp.float32),
  )(features, wmat.astype(jnp.bfloat16), bvec.reshape(1, kout))


def _make_gather_accumulate(f_vol, nout, n, aux):
  mesh = plsc.VectorSubcoreMesh(
      core_axis_name="c",
      subcore_axis_name="s",
      num_cores=_NUM_CORES,
      num_subcores=_NUM_SUBCORES,
  )

  rem = n % _CB
  nidx = f_vol + (1 if aux else 0)

  @functools.partial(
      pl.kernel,
      out_type=jax.ShapeDtypeStruct((n, nout), jnp.float32),
      mesh=mesh,
      scratch_types=[
          pltpu.VMEM((2, nidx, _CB), jnp.int32),
          pltpu.VMEM((2, _CB, nout), jnp.float32),
          pltpu.SemaphoreType.DMA,
          pltpu.SemaphoreType.DMA,
          pltpu.SemaphoreType.DMA,
          pltpu.SemaphoreType.DMA,
      ],
  )
  def gather_acc(*refs):
    if aux:
      t_hbm, a_hbm, idx_hbm, out_hbm, idx_v, acc_v, sg0, sg1, so0, so1 = refs
    else:
      t_hbm, idx_hbm, out_hbm, idx_v, acc_v, sg0, sg1, so0, so1 = refs
    wid = lax.axis_index("s") * _NUM_CORES + lax.axis_index("c")
    nch = _CHUNKS_PER_WORKER
    base_chunk = wid * _CHUNKS_PER_WORKER
    base_row = base_chunk * _CB
    sgs = (sg0, sg1)
    sos = (so0, so1)
    zeros = jnp.zeros((_LANES,), jnp.float32)

    def zero_acc(b):
      def zrow(r, carry):
        for k in range(nout // _LANES):
          acc_v[b, r, pl.ds(k * _LANES, _LANES)] = zeros
        return carry
      lax.fori_loop(0, _CB, zrow, 0)

    def fire_chunk(b, cc):
      pltpu.sync_copy(idx_hbm.at[base_chunk + cc], idx_v.at[b])
      for f in range(f_vol):
        pltpu.async_copy(
            t_hbm.at[idx_v.at[b, f]], acc_v.at[b], sgs[b], add=True)
      if aux:
        pltpu.async_copy(
            a_hbm.at[idx_v.at[b, f_vol]], acc_v.at[b], sgs[b], add=True)

    def drain_chunk(b):
      for f in range(f_vol):
        pltpu.make_async_copy(
            t_hbm.at[idx_v.at[b, f]], acc_v.at[b], sgs[b]).wait()
      if aux:
        pltpu.make_async_copy(
            a_hbm.at[idx_v.at[b, f_vol]], acc_v.at[b], sgs[b]).wait()

    zero_acc(0)
    zero_acc(1)
    fire_chunk(0, 0)
    fire_chunk(1, 1)

    def step(g, carry):
      for b in range(2):
        cc = 2 * g + b
        drain_chunk(b)
        off = base_row + cc * _CB
        @pl.when(off + _CB <= n)
        def _full():
          pltpu.async_copy(
              acc_v.at[b], out_hbm.at[pl.ds(off, _CB)], sos[b]).wait()
        if rem:
          @pl.when(off == n - rem)
          def _partial():
            pltpu.async_copy(
                acc_v.at[b, pl.ds(0, rem)],
                out_hbm.at[pl.ds(n - rem, rem)], sos[b]).wait()
        @pl.when(cc + 2 < nch)
        def _prep():
          zero_acc(b)
          fire_chunk(b, cc + 2)
      return carry

    lax.fori_loop(0, nch // 2, step, 0)

  return gather_acc


def _chunked_idx(cols):
  k = cols.shape[0]
  return cols.reshape(k, _N_CHUNKS, _CB).transpose(1, 0, 2)


def kernel(features, neighbor_idx, weight, bias):
  n, nin = features.shape
  f_vol = weight.shape[0]
  nout = weight.shape[2]
  ka = _SPLIT
  kb = f_vol - ka

  wmat = weight.transpose(1, 0, 2).reshape(nin, f_vol * nout)
  bvec_a = jnp.concatenate(
      [bias, jnp.zeros(((ka - 1) * nout,), jnp.float32)])
  tab_a = _matmul_tables(features, wmat[:, :ka * nout], bvec_a)
  tab_b = _matmul_tables(features, wmat[:, ka * nout:],
                         jnp.zeros((kb * nout,), jnp.float32))

  sites = neighbor_idx.T.astype(jnp.int32)
  sites = jnp.pad(sites, ((0, 0), (0, _N_PAD - n)))
  foff = (jnp.arange(f_vol, dtype=jnp.int32) * n)[:, None]
  idx_a = _chunked_idx(sites[:ka] + foff[:ka])
  ident = jnp.where(jnp.arange(_N_PAD, dtype=jnp.int32) < n,
                    jnp.arange(_N_PAD, dtype=jnp.int32), 0)
  idx_b = _chunked_idx(
      jnp.concatenate([sites[ka:] + foff[:kb], ident[None, :]], axis=0))

  out_a = _make_gather_accumulate(ka, nout, n, False)(
      tab_a.reshape(ka * n, nout), idx_a)
  return _make_gather_accumulate(kb, nout, n, True)(
      tab_b.reshape(kb * n, nout), out_a, idx_b)


# speedup vs baseline: 1.0270x; 1.0270x over previous
import functools

import jax
import jax.numpy as jnp
from jax import lax
from jax.experimental import pallas as pl
from jax.experimental.pallas import tpu as pltpu
from jax.experimental.pallas import tpu_sc as plsc

_NUM_CORES = 2
_NUM_SUBCORES = 16
_NUM_WORKERS = _NUM_CORES * _NUM_SUBCORES

_CB = 112
_N_CHUNKS = 448
_N_PAD = _CB * _N_CHUNKS
_CHUNKS_PER_WORKER = _N_CHUNKS // _NUM_WORKERS
_LANES = 16

_SPLIT = 4


def _matmul_tables(features, wmat, bvec):
  n, nin = features.shape
  kout = wmat.shape[1]
  bn = 5000
  assert n % bn == 0

  f_vol = kout // nin

  def body(x_ref, w_ref, b_ref, t_ref):
    acc = (
        jnp.dot(x_ref[...].astype(jnp.bfloat16), w_ref[...],
                preferred_element_type=jnp.float32)
        + b_ref[...])
    for f in range(f_vol):
      t_ref[f] = acc[:, f * nin:(f + 1) * nin]

  return pl.pallas_call(
      body,
      grid=(n // bn,),
      in_specs=[
          pl.BlockSpec((bn, nin), lambda i: (i, 0)),
          pl.BlockSpec((nin, kout), lambda i: (0, 0)),
          pl.BlockSpec((1, kout), lambda i: (0, 0)),
      ],
      out_specs=pl.BlockSpec((f_vol, bn, nin), lambda i: (0, i, 0)),
      out_shape=jax.ShapeDtypeStruct((f_vol, n, nin), jnp.float32),
  )(features, wmat.astype(jnp.bfloat16), bvec.reshape(1, kout))


def _make_gather_accumulate(f_vol, nout, n, aux):
  mesh = plsc.VectorSubcoreMesh(
      core_axis_name="c",
      subcore_axis_name="s",
      num_cores=_NUM_CORES,
      num_subcores=_NUM_SUBCORES,
  )

  rem = n % _CB
  nidx = f_vol + (1 if aux else 0)

  @functools.partial(
      pl.kernel,
      out_type=jax.ShapeDtypeStruct((n, nout), jnp.float32),
      mesh=mesh,
      scratch_types=[
          pltpu.VMEM((2, nidx, _CB), jnp.int32),
          pltpu.VMEM((2, _CB, nout), jnp.float32),
          pltpu.SemaphoreType.DMA,
          pltpu.SemaphoreType.DMA,
          pltpu.SemaphoreType.DMA,
          pltpu.SemaphoreType.DMA,
      ],
  )
  def gather_acc(*refs):
    if aux:
      t_hbm, a_hbm, idx_hbm, out_hbm, idx_v, acc_v, sg0, sg1, so0, so1 = refs
    else:
      t_hbm, idx_hbm, out_hbm, idx_v, acc_v, sg0, sg1, so0, so1 = refs
    wid = lax.axis_index("s") * _NUM_CORES + lax.axis_index("c")
    nch = _CHUNKS_PER_WORKER
    base_chunk = wid * _CHUNKS_PER_WORKER
    base_row = base_chunk * _CB
    sgs = (sg0, sg1)
    sos = (so0, so1)
    zeros = jnp.zeros((_LANES,), jnp.float32)

    def zero_acc(b):
      def zrow(r, carry):
        for k in range(nout // _LANES):
          acc_v[b, r, pl.ds(k * _LANES, _LANES)] = zeros
        return carry
      lax.fori_loop(0, _CB, zrow, 0)

    def fire_chunk(b, cc):
      pltpu.sync_copy(idx_hbm.at[base_chunk + cc], idx_v.at[b])
      for f in range(f_vol):
        pltpu.async_copy(
            t_hbm.at[idx_v.at[b, f]], acc_v.at[b], sgs[b], add=True)
      if aux:
        pltpu.async_copy(
            a_hbm.at[idx_v.at[b, f_vol]], acc_v.at[b], sgs[b], add=True)

    def drain_chunk(b):
      for f in range(f_vol):
        pltpu.make_async_copy(
            t_hbm.at[idx_v.at[b, f]], acc_v.at[b], sgs[b]).wait()
      if aux:
        pltpu.make_async_copy(
            a_hbm.at[idx_v.at[b, f_vol]], acc_v.at[b], sgs[b]).wait()

    zero_acc(0)
    zero_acc(1)
    fire_chunk(0, 0)
    fire_chunk(1, 1)

    def step(g, carry):
      for b in range(2):
        cc = 2 * g + b
        drain_chunk(b)
        off = base_row + cc * _CB
        @pl.when(off + _CB <= n)
        def _full():
          pltpu.async_copy(
              acc_v.at[b], out_hbm.at[pl.ds(off, _CB)], sos[b]).wait()
        if rem:
          @pl.when(off == n - rem)
          def _partial():
            pltpu.async_copy(
                acc_v.at[b, pl.ds(0, rem)],
                out_hbm.at[pl.ds(n - rem, rem)], sos[b]).wait()
        @pl.when(cc + 2 < nch)
        def _prep():
          zero_acc(b)
          fire_chunk(b, cc + 2)
      return carry

    lax.fori_loop(0, nch // 2, step, 0)

  return gather_acc


def _chunked_idx(cols):
  k = cols.shape[0]
  return cols.reshape(k, _N_CHUNKS, _CB).transpose(1, 0, 2)


def kernel(features, neighbor_idx, weight, bias):
  n, nin = features.shape
  f_vol = weight.shape[0]
  nout = weight.shape[2]
  ka = _SPLIT
  kb = f_vol - ka

  wmat = weight.transpose(1, 0, 2).reshape(nin, f_vol * nout)
  bvec_a = jnp.concatenate(
      [bias, jnp.zeros(((ka - 1) * nout,), jnp.float32)])
  tab_a = _matmul_tables(features, wmat[:, :ka * nout], bvec_a)
  tab_b = _matmul_tables(features, wmat[:, ka * nout:],
                         jnp.zeros((kb * nout,), jnp.float32))

  sites = neighbor_idx.T.astype(jnp.int32)
  sites = jnp.pad(sites, ((0, 0), (0, _N_PAD - n)))
  foff = (jnp.arange(f_vol, dtype=jnp.int32) * n)[:, None]
  idx_a = _chunked_idx(sites[:ka] + foff[:ka])
  ident = jnp.where(jnp.arange(_N_PAD, dtype=jnp.int32) < n,
                    jnp.arange(_N_PAD, dtype=jnp.int32), 0)
  idx_b = _chunked_idx(
      jnp.concatenate([sites[ka:] + foff[:kb], ident[None, :]], axis=0))

  out_a = _make_gather_accumulate(ka, nout, n, False)(
      tab_a.reshape(ka * n, nout), idx_a)
  return _make_gather_accumulate(kb, nout, n, True)(
      tab_b.reshape(kb * n, nout), out_a, idx_b)
